# baseline (speedup 1.0000x reference)
import functools
import jax
import jax.numpy as jnp
from jax import lax
from jax.experimental import pallas as pl
from jax.experimental.pallas import tpu as pltpu
from jax.experimental.pallas import tpu_sc as plsc

NC = 2
NS = 16
NW = NC * NS
CH = 128

def _mesh():
  return plsc.VectorSubcoreMesh(core_axis_name="c", subcore_axis_name="s",
                                num_cores=NC, num_subcores=NS)


def _make_stats(n, kt):
  @functools.partial(
      pl.kernel,
      mesh=_mesh(),
      compiler_params=pltpu.CompilerParams(use_tc_tiling_on_sc=False),
      out_type=(
          jax.ShapeDtypeStruct((NC, n), jnp.float32),
          jax.ShapeDtypeStruct((NC, n), jnp.float32),
          jax.ShapeDtypeStruct((NC, n, 4), jnp.float32),
      ),
      scratch_types=[
          pltpu.VMEM_SHARED((n,), jnp.float32),
          pltpu.VMEM_SHARED((n,), jnp.float32),
          pltpu.VMEM_SHARED((n, 4), jnp.float32),
          pltpu.VMEM((kt, CH), jnp.int32),
          pltpu.VMEM((kt, CH), jnp.int32),
          pltpu.VMEM((kt, CH, 4), jnp.float32),
          pltpu.VMEM((kt, CH), jnp.float32),
      ],
  )
  def stats(src_h, dst_h, e_h, ones_h, z1_h, z4_h,
            degs_o, degd_o, esum_o,
            degs_sh, degd_sh, esum_sh, src_v, dst_v, e_v, ones_v):
    c = lax.axis_index("c")
    s = lax.axis_index("s")
    wid = c * NS + s
    rp = n // NS
    r0 = s * rp
    pltpu.sync_copy(z1_h.at[pl.ds(r0, rp)], degs_sh.at[pl.ds(r0, rp)])
    pltpu.sync_copy(z1_h.at[pl.ds(r0, rp)], degd_sh.at[pl.ds(r0, rp)])
    pltpu.sync_copy(z4_h.at[pl.ds(r0, rp)], esum_sh.at[pl.ds(r0, rp)])
    pltpu.sync_copy(src_h.at[pl.ds(wid * kt, kt)], src_v)
    pltpu.sync_copy(dst_h.at[pl.ds(wid * kt, kt)], dst_v)
    pltpu.sync_copy(e_h.at[pl.ds(wid * kt, kt)], e_v)
    pltpu.sync_copy(ones_h.at[pl.ds(wid * kt, kt)], ones_v)
    plsc.subcore_barrier()

    def chunk(j, carry):
      pltpu.sync_copy(ones_v.at[j], degs_sh.at[src_v.at[j]], add=True)
      pltpu.sync_copy(ones_v.at[j], degd_sh.at[dst_v.at[j]], add=True)
      pltpu.sync_copy(e_v.at[j], esum_sh.at[dst_v.at[j]], add=True)
      return carry

    lax.fori_loop(0, kt, chunk, 0)
    plsc.subcore_barrier()
    pltpu.sync_copy(degs_sh.at[pl.ds(r0, rp)], degs_o.at[c, pl.ds(r0, rp)])
    pltpu.sync_copy(degd_sh.at[pl.ds(r0, rp)], degd_o.at[c, pl.ds(r0, rp)])
    pltpu.sync_copy(esum_sh.at[pl.ds(r0, rp)], esum_o.at[c, pl.ds(r0, rp)])

  return stats


def _make_spmm(n, d, kt):
  @functools.partial(
      pl.kernel,
      mesh=_mesh(),
      compiler_params=pltpu.CompilerParams(use_tc_tiling_on_sc=False),
      out_type=jax.ShapeDtypeStruct((NC, n, d), jnp.float32),
      scratch_types=[
          pltpu.VMEM_SHARED((n, d), jnp.float32),
          pltpu.VMEM((kt, CH), jnp.float32),
          pltpu.VMEM((4, 2, CH), jnp.int32),
          pltpu.VMEM((CH, d), jnp.float32),
          pltpu.VMEM((CH, d), jnp.float32),
          pltpu.SemaphoreType.DMA,
          pltpu.SemaphoreType.DMA,
          pltpu.SemaphoreType.DMA,
          pltpu.SemaphoreType.DMA,
          pltpu.SemaphoreType.DMA,
          pltpu.SemaphoreType.DMA,
          pltpu.SemaphoreType.DMA,
          pltpu.SemaphoreType.DMA,
      ],
  )
  def spmm(h_h, sd_h, att_h, z_h,
           acc_o,
           acc_sh, att_v, sdr, rows0, rows1,
           gsem0, gsem1, ssem0, ssem1, isem0, isem1, isem2, isem3):
    c = lax.axis_index("c")
    s = lax.axis_index("s")
    wid = c * NS + s
    rp = n // NS
    r0 = s * rp
    cb = wid * kt
    pltpu.sync_copy(z_h.at[pl.ds(r0, rp)], acc_sh.at[pl.ds(r0, rp)])
    pltpu.sync_copy(att_h.at[pl.ds(cb, kt)], att_v)
    plsc.subcore_barrier()

    rows = (rows0, rows1)
    gsem = (gsem0, gsem1)
    ssem = (ssem0, ssem1)
    isem = (isem0, isem1, isem2, isem3)

    def src_at(sl):
      return sdr.at[sl, 0]

    def dst_at(sl):
      return sdr.at[sl, 1]

    pltpu.async_copy(sd_h.at[cb], sdr.at[0], isem0)
    pltpu.async_copy(sd_h.at[cb + 1], sdr.at[1], isem1)
    pltpu.make_async_copy(sd_h.at[cb], sdr.at[0], isem0).wait()
    pltpu.async_copy(h_h.at[src_at(0)], rows0, gsem0)

    def quad(p, carry):
      for u in range(4):
        j = 4 * p + u
        b = u % 2
        sl = u % 4
        rb, gs, ss = rows[b], gsem[b], ssem[b]
        nb, ngs, nss = rows[1 - b], gsem[1 - b], ssem[1 - b]

        @pl.when(j + 2 < kt)
        def _():
          pltpu.async_copy(sd_h.at[cb + j + 2], sdr.at[(sl + 2) % 4],
                           isem[(sl + 2) % 4])

        @pl.when(j + 1 < kt)
        def _():
          @pl.when(j >= 1)
          def _():
            pltpu.make_async_copy(nb, acc_sh.at[dst_at((sl + 3) % 4)],
                                  nss).wait()

          pltpu.make_async_copy(sd_h.at[cb + j + 1], sdr.at[(sl + 1) % 4],
                                isem[(sl + 1) % 4]).wait()
          pltpu.async_copy(h_h.at[src_at((sl + 1) % 4)], nb, ngs)

        pltpu.make_async_copy(h_h.at[src_at(sl)], rb, gs).wait()

        def grp16(g, cc):
          av = att_v[j, pl.ds(g * 16, 16)]
          for k in range(16):
            a = av[k]
            i = g * 16 + k
            for q in range(d // 16):
              rb[i, pl.ds(q * 16, 16)] = rb[i, pl.ds(q * 16, 16)] * a
          return cc

        lax.fori_loop(0, CH // 16, grp16, 0)
        pltpu.async_copy(rb, acc_sh.at[dst_at(sl)], ss, add=True)
      return carry

    lax.fori_loop(0, kt // 4, quad, 0)
    pltpu.make_async_copy(rows0, acc_sh.at[dst_at(2)], ssem0).wait()
    pltpu.make_async_copy(rows1, acc_sh.at[dst_at(3)], ssem1).wait()
    plsc.subcore_barrier()
    pltpu.sync_copy(acc_sh.at[pl.ds(r0, rp)], acc_o.at[c, pl.ds(r0, rp)])

  return spmm


def _make_decode(n, lp):
  per = lp // NW

  @functools.partial(
      pl.kernel,
      mesh=_mesh(),
      compiler_params=pltpu.CompilerParams(use_tc_tiling_on_sc=False,
                                           needs_layout_passes=False),
      out_type=jax.ShapeDtypeStruct((lp,), jnp.float32),
      scratch_types=[
          pltpu.VMEM((n,), jnp.float32),
          pltpu.VMEM((per,), jnp.int32),
          pltpu.VMEM((per,), jnp.int32),
          pltpu.VMEM((per,), jnp.float32),
      ],
  )
  def decode(pred_h, li0_h, li1_h, out_o, pred_v, a_v, b_v, o_v):
    c = lax.axis_index("c")
    s = lax.axis_index("s")
    wid = c * NS + s
    pltpu.sync_copy(pred_h, pred_v)
    pltpu.sync_copy(li0_h.at[pl.ds(wid * per, per)], a_v)
    pltpu.sync_copy(li1_h.at[pl.ds(wid * per, per)], b_v)

    def grp(g, carry):
      ia = a_v[pl.ds(g * 16, 16)]
      ib = b_v[pl.ds(g * 16, 16)]
      va = plsc.load_gather(pred_v, [ia])
      vb = plsc.load_gather(pred_v, [ib])
      o_v[pl.ds(g * 16, 16)] = va * vb
      return carry

    lax.fori_loop(0, per // 16, grp, 0)
    pltpu.sync_copy(o_v, out_o.at[pl.ds(wid * per, per)])

  return decode


def _prelude_body(e_r, x, w, b, degs_p, degd_p, esum_p,
                  att_o, h0_o, dinv_o, degd_o, combo_o):
  deg = degs_p[0] + degs_p[1]
  dinv = jnp.where(deg > 0.0, lax.rsqrt(deg), 0.0)
  dinv_o[...] = dinv
  h0_o[...] = x[...] * dinv
  degd = degd_p[0] + degd_p[1]
  degd_o[...] = degd
  esum = esum_p[0] + esum_p[1]
  combo_o[...] = jnp.concatenate(
      [esum, degd, jnp.zeros_like(esum[:, :3])], axis=-1)
  logits = jnp.dot(e_r[...], w[...], preferred_element_type=jnp.float32)
  att_o[...] = jax.nn.sigmoid(logits + b[...])


def _prelude(e_r, x, w, b, degs_p, degd_p, esum_p, rb):
  n = x.shape[0]
  grid = n // rb
  return pl.pallas_call(
      _prelude_body,
      grid=(grid,),
      in_specs=[
          pl.BlockSpec((rb, e_r.shape[1]), lambda i: (i, 0)),
          pl.BlockSpec((rb, x.shape[1]), lambda i: (i, 0)),
          pl.BlockSpec(w.shape, lambda i: (0, 0)),
          pl.BlockSpec(b.shape, lambda i: (0, 0)),
          pl.BlockSpec((NC, rb, 1), lambda i: (0, i, 0)),
          pl.BlockSpec((NC, rb, 1), lambda i: (0, i, 0)),
          pl.BlockSpec((NC, rb, 4), lambda i: (0, i, 0)),
      ],
      out_specs=[
          pl.BlockSpec((rb, w.shape[1]), lambda i: (i, 0)),
          pl.BlockSpec((rb, x.shape[1]), lambda i: (i, 0)),
          pl.BlockSpec((rb, 1), lambda i: (i, 0)),
          pl.BlockSpec((rb, 1), lambda i: (i, 0)),
          pl.BlockSpec((rb, 8), lambda i: (i, 0)),
      ],
      out_shape=[
          jax.ShapeDtypeStruct((n, w.shape[1]), jnp.float32),
          jax.ShapeDtypeStruct((n, x.shape[1]), jnp.float32),
          jax.ShapeDtypeStruct((n, 1), jnp.float32),
          jax.ShapeDtypeStruct((n, 1), jnp.float32),
          jax.ShapeDtypeStruct((n, 8), jnp.float32),
      ],
  )(e_r, x, w, b, degs_p, degd_p, esum_p)


def _dense_body(final, x, h, acc_p, combo, degd, dinv, wc, msg_wt, msg_b,
                skip_wt, skip_b, w1t, w2v, out_o, hn_o):
  acc = acc_p[0] + acc_p[1]
  rst = acc + jnp.dot(combo[...], wc[...], preferred_element_type=jnp.float32)
  rst = rst + degd[...] * h[...]
  rst = rst * dinv[...]
  out = (jnp.dot(rst, msg_wt[...], preferred_element_type=jnp.float32) + msg_b[...]
         + jnp.dot(x[...], skip_wt[...], preferred_element_type=jnp.float32)
         + skip_b[...])
  if not final:
    out_o[...] = out
    hn_o[...] = out * dinv[...]
  else:
    nrm = jnp.maximum(jnp.sqrt(jnp.sum(out * out, axis=-1, keepdims=True)), 1e-12)
    hn = out / nrm
    p1 = jnp.maximum(
        jnp.dot(hn, w1t[...], preferred_element_type=jnp.float32), 0.0)
    pred = jax.nn.sigmoid(jnp.sum(p1 * w2v[...], axis=-1, keepdims=True))
    out_o[...] = pred
    hn_o[...] = hn


def _dense(final, x, h, acc_p, combo, degd, dinv, wc, msg_wt, msg_b,
           skip_wt, skip_b, w1t, w2v, rb):
  n, d = x.shape
  grid = n // rb
  full = lambda a: pl.BlockSpec(a.shape, lambda i: tuple(0 for _ in a.shape))
  out_w = 1 if final else d
  return pl.pallas_call(
      functools.partial(_dense_body, final),
      grid=(grid,),
      in_specs=[
          pl.BlockSpec((rb, d), lambda i: (i, 0)),
          pl.BlockSpec((rb, d), lambda i: (i, 0)),
          pl.BlockSpec((NC, rb, d), lambda i: (0, i, 0)),
          pl.BlockSpec((rb, 8), lambda i: (i, 0)),
          pl.BlockSpec((rb, 1), lambda i: (i, 0)),
          pl.BlockSpec((rb, 1), lambda i: (i, 0)),
          full(wc), full(msg_wt), full(msg_b), full(skip_wt), full(skip_b),
          full(w1t), full(w2v),
      ],
      out_specs=[
          pl.BlockSpec((rb, out_w), lambda i: (i, 0)),
          pl.BlockSpec((rb, d), lambda i: (i, 0)),
      ],
      out_shape=[
          jax.ShapeDtypeStruct((n, out_w), jnp.float32),
          jax.ShapeDtypeStruct((n, d), jnp.float32),
      ],
  )(x, h, acc_p, combo, degd, dinv, wc, msg_wt, msg_b, skip_wt, skip_b,
    w1t, w2v)


def kernel(x, e, edge_index, label_edge_index,
           skip_w0, skip_b0, msg_w0, msg_b0, mlp_w0, mlp_b0, att_w0, att_b0,
           skip_w1, skip_b1, msg_w1, msg_b1, mlp_w1, mlp_b1, att_w1, att_b1,
           weight1, weight2):
  n, d = x.shape
  eN, de = e.shape
  l = label_edge_index.shape[1]

  src = edge_index[0]
  dst = edge_index[1]

  rbn = 1024
  np_ = -(-n // rbn) * rbn
  x_p = jnp.concatenate([x, jnp.zeros((np_ - n, d), jnp.float32)])

  grp = NW * CH
  kt = -(-(-(-eN // grp)) // 8) * 8
  e_pad = grp * kt
  pad = e_pad - eN
  src_p = jnp.concatenate([src, jnp.zeros((pad,), jnp.int32)]).reshape(-1, CH)
  dst_p = jnp.concatenate([dst, jnp.zeros((pad,), jnp.int32)]).reshape(-1, CH)
  e_p = jnp.concatenate([e, jnp.zeros((pad, de), jnp.float32)]).reshape(-1, CH, de)
  ones_p = jnp.concatenate(
      [jnp.ones((eN,), jnp.float32), jnp.zeros((pad,), jnp.float32)]
  ).reshape(-1, CH)
  z_n1 = jnp.zeros((np_,), jnp.float32)
  z_n4 = jnp.zeros((np_, 4), jnp.float32)
  z_nd = jnp.zeros((np_, d), jnp.float32)

  degs_p, degd_p, esum_p = _make_stats(np_, kt)(src_p, dst_p, e_p, ones_p,
                                                z_n1, z_n4)
  degs_p = degs_p.reshape(NC, np_, 1)
  degd_p = degd_p.reshape(NC, np_, 1)

  epr = 128 // de
  ner = eN // epr
  e_r = e.reshape(ner, epr * de)
  e_r = jnp.concatenate(
      [e_r, jnp.zeros((np_ - ner, epr * de), jnp.float32)])
  wa = jnp.concatenate(
      [jnp.kron(jnp.eye(epr, dtype=jnp.float32), att_w0.reshape(de, 1)),
       jnp.kron(jnp.eye(epr, dtype=jnp.float32), att_w1.reshape(de, 1))],
      axis=1)
  ba = jnp.concatenate([jnp.tile(att_b0, epr), jnp.tile(att_b1, epr)])[None, :]
  att2, h0, dinv, degd, combo = _prelude(e_r, x_p, wa, ba, degs_p, degd_p,
                                         esum_p, rb=rbn)
  att0 = att2[:ner, :epr].reshape(eN)
  att1 = att2[:ner, epr:].reshape(eN)
  zpad = jnp.zeros((pad,), jnp.float32)
  att0_p = jnp.concatenate([att0, zpad]).reshape(-1, CH)
  att1_p = jnp.concatenate([att1, zpad]).reshape(-1, CH)

  def wpack(mlp_w, mlp_b):
    return jnp.concatenate(
        [mlp_w.T, mlp_b[None, :], jnp.zeros((3, d), jnp.float32)], axis=0)

  spmm = _make_spmm(np_, d, kt)
  sd_p = jnp.stack([src_p, dst_p], axis=1)

  acc0 = spmm(h0, sd_p, att0_p, z_nd)
  out0, h1 = _dense(False, x_p, h0, acc0, combo, degd, dinv,
                    wpack(mlp_w0, mlp_b0), msg_w0.T, msg_b0[None, :],
                    skip_w0.T, skip_b0[None, :], jnp.zeros((d, d), jnp.float32),
                    jnp.zeros((1, d), jnp.float32), rb=rbn)

  acc1 = spmm(h1, sd_p, att1_p, z_nd)
  pred, _ = _dense(True, out0, h1, acc1, combo, degd, dinv,
                   wpack(mlp_w1, mlp_b1), msg_w1.T, msg_b1[None, :],
                   skip_w1.T, skip_b1[None, :], weight1.T, weight2, rb=rbn)

  lgrp = NW * 16
  lkt = -(-l // lgrp)
  l_pad = lgrp * lkt
  lp0 = jnp.concatenate(
      [label_edge_index[0], jnp.zeros((l_pad - l,), jnp.int32)])
  lp1 = jnp.concatenate(
      [label_edge_index[1], jnp.zeros((l_pad - l,), jnp.int32)])
  res = _make_decode(np_, l_pad)(pred.reshape(np_), lp0, lp1)
  return res[:l]


# speedup vs baseline: 1.5500x; 1.5500x over previous
import functools
import jax
import jax.numpy as jnp
from jax import lax
from jax.experimental import pallas as pl
from jax.experimental.pallas import tpu as pltpu
from jax.experimental.pallas import tpu_sc as plsc

NC = 2
NS = 16
NW = NC * NS
CH = 64

def _mesh():
  return plsc.VectorSubcoreMesh(core_axis_name="c", subcore_axis_name="s",
                                num_cores=NC, num_subcores=NS)


def _make_stats(n, kt):
  @functools.partial(
      pl.kernel,
      mesh=_mesh(),
      compiler_params=pltpu.CompilerParams(use_tc_tiling_on_sc=False),
      out_type=(
          jax.ShapeDtypeStruct((NC, n), jnp.float32),
          jax.ShapeDtypeStruct((NC, n), jnp.float32),
          jax.ShapeDtypeStruct((NC, n, 4), jnp.float32),
      ),
      scratch_types=[
          pltpu.VMEM_SHARED((n,), jnp.float32),
          pltpu.VMEM_SHARED((n,), jnp.float32),
          pltpu.VMEM_SHARED((n, 4), jnp.float32),
          pltpu.VMEM((kt, CH), jnp.int32),
          pltpu.VMEM((kt, CH), jnp.int32),
          pltpu.VMEM((kt, CH, 4), jnp.float32),
          pltpu.VMEM((kt, CH), jnp.float32),
      ],
  )
  def stats(src_h, dst_h, e_h, ones_h, z1_h, z4_h,
            degs_o, degd_o, esum_o,
            degs_sh, degd_sh, esum_sh, src_v, dst_v, e_v, ones_v):
    c = lax.axis_index("c")
    s = lax.axis_index("s")
    wid = c * NS + s
    rp = n // NS
    r0 = s * rp
    pltpu.sync_copy(z1_h.at[pl.ds(r0, rp)], degs_sh.at[pl.ds(r0, rp)])
    pltpu.sync_copy(z1_h.at[pl.ds(r0, rp)], degd_sh.at[pl.ds(r0, rp)])
    pltpu.sync_copy(z4_h.at[pl.ds(r0, rp)], esum_sh.at[pl.ds(r0, rp)])
    pltpu.sync_copy(src_h.at[pl.ds(wid * kt, kt)], src_v)
    pltpu.sync_copy(dst_h.at[pl.ds(wid * kt, kt)], dst_v)
    pltpu.sync_copy(e_h.at[pl.ds(wid * kt, kt)], e_v)
    pltpu.sync_copy(ones_h.at[pl.ds(wid * kt, kt)], ones_v)
    plsc.subcore_barrier()

    def chunk(j, carry):
      pltpu.sync_copy(ones_v.at[j], degs_sh.at[src_v.at[j]], add=True)
      pltpu.sync_copy(ones_v.at[j], degd_sh.at[dst_v.at[j]], add=True)
      pltpu.sync_copy(e_v.at[j], esum_sh.at[dst_v.at[j]], add=True)
      return carry

    lax.fori_loop(0, kt, chunk, 0)
    plsc.subcore_barrier()
    pltpu.sync_copy(degs_sh.at[pl.ds(r0, rp)], degs_o.at[c, pl.ds(r0, rp)])
    pltpu.sync_copy(degd_sh.at[pl.ds(r0, rp)], degd_o.at[c, pl.ds(r0, rp)])
    pltpu.sync_copy(esum_sh.at[pl.ds(r0, rp)], esum_o.at[c, pl.ds(r0, rp)])

  return stats


def _make_spmm(n, d, kt):
  d2 = d // 2

  @functools.partial(
      pl.kernel,
      mesh=_mesh(),
      compiler_params=pltpu.CompilerParams(use_tc_tiling_on_sc=False,
                                           needs_layout_passes=False),
      out_type=jax.ShapeDtypeStruct((n, d), jnp.float32),
      scratch_types=[
          pltpu.VMEM_SHARED((n, d2), jnp.float32),
          pltpu.VMEM_SHARED((n, d2), jnp.float32),
          pltpu.VMEM((kt, CH), jnp.int32),
          pltpu.VMEM((kt, CH), jnp.bfloat16),
          pltpu.VMEM((2, CH), jnp.int32),
          pltpu.VMEM((2, CH), jnp.int32),
          pltpu.VMEM((CH, d2), jnp.float32),
          pltpu.VMEM((CH, d2), jnp.float32),
          pltpu.SemaphoreType.DMA,
          pltpu.SemaphoreType.DMA,
          pltpu.SemaphoreType.DMA,
          pltpu.SemaphoreType.DMA,
      ],
  )
  def spmm(h_h, sdp_h, attb_h, z_h,
           acc_o,
           h_sh, acc_sh, sdp_v, attb_v, u0, u1, rows0, rows1,
           gsem0, gsem1, ssem0, ssem1):
    c = lax.axis_index("c")
    s = lax.axis_index("s")
    rp = n // NS
    r0 = s * rp
    c64 = c * d2
    pltpu.sync_copy(h_h.at[pl.ds(r0, rp), pl.ds(c64, d2)], h_sh.at[pl.ds(r0, rp)])
    pltpu.sync_copy(z_h.at[pl.ds(r0, rp), pl.ds(0, d2)], acc_sh.at[pl.ds(r0, rp)])
    cb = s * kt
    pltpu.sync_copy(sdp_h.at[pl.ds(cb, kt)], sdp_v)
    pltpu.sync_copy(attb_h.at[pl.ds(cb, kt)], attb_v)
    plsc.subcore_barrier()

    rows = (rows0, rows1)
    gsem = (gsem0, gsem1)
    ssem = (ssem0, ssem1)
    ubuf = (u0, u1)

    def unpack(j, ub):
      for g in range(CH // 16):
        v = sdp_v[j, pl.ds(g * 16, 16)]
        ub[0, pl.ds(g * 16, 16)] = v & 0xFFFF
        ub[1, pl.ds(g * 16, 16)] = lax.shift_right_logical(v, 16)

    unpack(0, u0)
    pltpu.async_copy(h_sh.at[u0.at[0]], rows0, gsem0)

    def pair(p, carry):
      for b in range(2):
        j = 2 * p + b
        rb, gs, ss, ub = rows[b], gsem[b], ssem[b], ubuf[b]
        nb, ngs, nss, nub = rows[1 - b], gsem[1 - b], ssem[1 - b], ubuf[1 - b]

        @pl.when(j + 1 < kt)
        def _():
          @pl.when(j >= 1)
          def _():
            pltpu.make_async_copy(nb, acc_sh.at[nub.at[1]], nss).wait()

          unpack(j + 1, nub)
          pltpu.async_copy(h_sh.at[nub.at[0]], nb, ngs)

        pltpu.make_async_copy(h_sh.at[ub.at[0]], rb, gs).wait()

        def grp32(g, cc):
          bv = plsc.bitcast(attb_v[j, pl.ds(g * 32, 32)], jnp.int32)
          for m in range(16):
            wm = bv[m]
            a0 = lax.bitcast_convert_type(lax.shift_left(wm, 16), jnp.float32)
            a1 = lax.bitcast_convert_type(wm & (-65536), jnp.float32)
            for t, a in ((0, a0), (1, a1)):
              i = g * 32 + 2 * m + t
              for q in range(d2 // 16):
                rb[i, pl.ds(q * 16, 16)] = rb[i, pl.ds(q * 16, 16)] * a
          return cc

        lax.fori_loop(0, CH // 32, grp32, 0)
        pltpu.async_copy(rb, acc_sh.at[ub.at[1]], ss, add=True)
      return carry

    lax.fori_loop(0, kt // 2, pair, 0)
    pltpu.make_async_copy(rows0, acc_sh.at[u0.at[1]], ssem0).wait()
    pltpu.make_async_copy(rows1, acc_sh.at[u1.at[1]], ssem1).wait()
    plsc.subcore_barrier()
    pltpu.sync_copy(acc_sh.at[pl.ds(r0, rp)],
                    acc_o.at[pl.ds(r0, rp), pl.ds(c64, d2)])

  return spmm


def _make_decode(n, lp):
  per = lp // NW

  @functools.partial(
      pl.kernel,
      mesh=_mesh(),
      compiler_params=pltpu.CompilerParams(use_tc_tiling_on_sc=False,
                                           needs_layout_passes=False),
      out_type=jax.ShapeDtypeStruct((lp,), jnp.float32),
      scratch_types=[
          pltpu.VMEM((n,), jnp.float32),
          pltpu.VMEM((per,), jnp.int32),
          pltpu.VMEM((per,), jnp.int32),
          pltpu.VMEM((per,), jnp.float32),
      ],
  )
  def decode(pred_h, li0_h, li1_h, out_o, pred_v, a_v, b_v, o_v):
    c = lax.axis_index("c")
    s = lax.axis_index("s")
    wid = c * NS + s
    pltpu.sync_copy(pred_h, pred_v)
    pltpu.sync_copy(li0_h.at[pl.ds(wid * per, per)], a_v)
    pltpu.sync_copy(li1_h.at[pl.ds(wid * per, per)], b_v)

    def grp(g, carry):
      ia = a_v[pl.ds(g * 16, 16)]
      ib = b_v[pl.ds(g * 16, 16)]
      va = plsc.load_gather(pred_v, [ia])
      vb = plsc.load_gather(pred_v, [ib])
      o_v[pl.ds(g * 16, 16)] = va * vb
      return carry

    lax.fori_loop(0, per // 16, grp, 0)
    pltpu.sync_copy(o_v, out_o.at[pl.ds(wid * per, per)])

  return decode


def _prelude_body(e_r, x, w, b, degs_p, degd_p, esum_p,
                  att_o, h0_o, dinv_o, degd_o, combo_o):
  deg = degs_p[0] + degs_p[1]
  dinv = jnp.where(deg > 0.0, lax.rsqrt(deg), 0.0)
  dinv_o[...] = dinv
  h0_o[...] = x[...] * dinv
  degd = degd_p[0] + degd_p[1]
  degd_o[...] = degd
  esum = esum_p[0] + esum_p[1]
  combo_o[...] = jnp.concatenate(
      [esum, degd, jnp.zeros_like(esum[:, :3])], axis=-1)
  logits = jnp.dot(e_r[...], w[...], preferred_element_type=jnp.float32)
  att_o[...] = jax.nn.sigmoid(logits + b[...])


def _prelude(e_r, x, w, b, degs_p, degd_p, esum_p, rb):
  n = x.shape[0]
  grid = n // rb
  return pl.pallas_call(
      _prelude_body,
      grid=(grid,),
      in_specs=[
          pl.BlockSpec((rb, e_r.shape[1]), lambda i: (i, 0)),
          pl.BlockSpec((rb, x.shape[1]), lambda i: (i, 0)),
          pl.BlockSpec(w.shape, lambda i: (0, 0)),
          pl.BlockSpec(b.shape, lambda i: (0, 0)),
          pl.BlockSpec((NC, rb, 1), lambda i: (0, i, 0)),
          pl.BlockSpec((NC, rb, 1), lambda i: (0, i, 0)),
          pl.BlockSpec((NC, rb, 4), lambda i: (0, i, 0)),
      ],
      out_specs=[
          pl.BlockSpec((rb, w.shape[1]), lambda i: (i, 0)),
          pl.BlockSpec((rb, x.shape[1]), lambda i: (i, 0)),
          pl.BlockSpec((rb, 1), lambda i: (i, 0)),
          pl.BlockSpec((rb, 1), lambda i: (i, 0)),
          pl.BlockSpec((rb, 8), lambda i: (i, 0)),
      ],
      out_shape=[
          jax.ShapeDtypeStruct((n, w.shape[1]), jnp.float32),
          jax.ShapeDtypeStruct((n, x.shape[1]), jnp.float32),
          jax.ShapeDtypeStruct((n, 1), jnp.float32),
          jax.ShapeDtypeStruct((n, 1), jnp.float32),
          jax.ShapeDtypeStruct((n, 8), jnp.float32),
      ],
  )(e_r, x, w, b, degs_p, degd_p, esum_p)


def _dense_body(final, x, h, acc_p, combo, degd, dinv, wc, msg_wt, msg_b,
                skip_wt, skip_b, w1t, w2v, out_o, hn_o):
  acc = acc_p[...]
  rst = acc + jnp.dot(combo[...], wc[...], preferred_element_type=jnp.float32)
  rst = rst + degd[...] * h[...]
  rst = rst * dinv[...]
  out = (jnp.dot(rst, msg_wt[...], preferred_element_type=jnp.float32) + msg_b[...]
         + jnp.dot(x[...], skip_wt[...], preferred_element_type=jnp.float32)
         + skip_b[...])
  if not final:
    out_o[...] = out
    hn_o[...] = out * dinv[...]
  else:
    nrm = jnp.maximum(jnp.sqrt(jnp.sum(out * out, axis=-1, keepdims=True)), 1e-12)
    hn = out / nrm
    p1 = jnp.maximum(
        jnp.dot(hn, w1t[...], preferred_element_type=jnp.float32), 0.0)
    pred = jax.nn.sigmoid(jnp.sum(p1 * w2v[...], axis=-1, keepdims=True))
    out_o[...] = pred
    hn_o[...] = hn


def _dense(final, x, h, acc_p, combo, degd, dinv, wc, msg_wt, msg_b,
           skip_wt, skip_b, w1t, w2v, rb):
  n, d = x.shape
  grid = n // rb
  full = lambda a: pl.BlockSpec(a.shape, lambda i: tuple(0 for _ in a.shape))
  out_w = 1 if final else d
  return pl.pallas_call(
      functools.partial(_dense_body, final),
      grid=(grid,),
      in_specs=[
          pl.BlockSpec((rb, d), lambda i: (i, 0)),
          pl.BlockSpec((rb, d), lambda i: (i, 0)),
          pl.BlockSpec((rb, d), lambda i: (i, 0)),
          pl.BlockSpec((rb, 8), lambda i: (i, 0)),
          pl.BlockSpec((rb, 1), lambda i: (i, 0)),
          pl.BlockSpec((rb, 1), lambda i: (i, 0)),
          full(wc), full(msg_wt), full(msg_b), full(skip_wt), full(skip_b),
          full(w1t), full(w2v),
      ],
      out_specs=[
          pl.BlockSpec((rb, out_w), lambda i: (i, 0)),
          pl.BlockSpec((rb, d), lambda i: (i, 0)),
      ],
      out_shape=[
          jax.ShapeDtypeStruct((n, out_w), jnp.float32),
          jax.ShapeDtypeStruct((n, d), jnp.float32),
      ],
  )(x, h, acc_p, combo, degd, dinv, wc, msg_wt, msg_b, skip_wt, skip_b,
    w1t, w2v)


def kernel(x, e, edge_index, label_edge_index,
           skip_w0, skip_b0, msg_w0, msg_b0, mlp_w0, mlp_b0, att_w0, att_b0,
           skip_w1, skip_b1, msg_w1, msg_b1, mlp_w1, mlp_b1, att_w1, att_b1,
           weight1, weight2):
  n, d = x.shape
  eN, de = e.shape
  l = label_edge_index.shape[1]

  src = edge_index[0]
  dst = edge_index[1]

  rbn = 1024
  np_ = -(-n // rbn) * rbn
  x_p = jnp.concatenate([x, jnp.zeros((np_ - n, d), jnp.float32)])

  grp = NW * CH
  kt = -(-(-(-eN // grp)) // 8) * 8
  kt2 = 2 * kt
  e_pad = grp * kt
  pad = e_pad - eN
  src_p = jnp.concatenate([src, jnp.zeros((pad,), jnp.int32)]).reshape(-1, CH)
  dst_p = jnp.concatenate([dst, jnp.zeros((pad,), jnp.int32)]).reshape(-1, CH)
  e_p = jnp.concatenate([e, jnp.zeros((pad, de), jnp.float32)]).reshape(-1, CH, de)
  ones_p = jnp.concatenate(
      [jnp.ones((eN,), jnp.float32), jnp.zeros((pad,), jnp.float32)]
  ).reshape(-1, CH)
  z_n1 = jnp.zeros((np_,), jnp.float32)
  z_n4 = jnp.zeros((np_, 4), jnp.float32)
  z_nd = jnp.zeros((np_, d), jnp.float32)

  degs_p, degd_p, esum_p = _make_stats(np_, kt)(src_p, dst_p, e_p, ones_p,
                                                z_n1, z_n4)
  degs_p = degs_p.reshape(NC, np_, 1)
  degd_p = degd_p.reshape(NC, np_, 1)

  epr = 128 // de
  ner = eN // epr
  e_r = e.reshape(ner, epr * de)
  e_r = jnp.concatenate(
      [e_r, jnp.zeros((np_ - ner, epr * de), jnp.float32)])
  wa = jnp.concatenate(
      [jnp.kron(jnp.eye(epr, dtype=jnp.float32), att_w0.reshape(de, 1)),
       jnp.kron(jnp.eye(epr, dtype=jnp.float32), att_w1.reshape(de, 1))],
      axis=1)
  ba = jnp.concatenate([jnp.tile(att_b0, epr), jnp.tile(att_b1, epr)])[None, :]
  att2, h0, dinv, degd, combo = _prelude(e_r, x_p, wa, ba, degs_p, degd_p,
                                         esum_p, rb=rbn)
  att0 = att2[:ner, :epr].reshape(eN)
  att1 = att2[:ner, epr:].reshape(eN)
  zpad = jnp.zeros((pad,), jnp.float32)
  att0_p = jnp.concatenate([att0, zpad]).reshape(-1, CH)
  att1_p = jnp.concatenate([att1, zpad]).reshape(-1, CH)

  def wpack(mlp_w, mlp_b):
    return jnp.concatenate(
        [mlp_w.T, mlp_b[None, :], jnp.zeros((3, d), jnp.float32)], axis=0)

  spmm = _make_spmm(np_, d, kt2)
  sdp = (src_p + dst_p * 65536).astype(jnp.int32)
  attb0 = att0_p.astype(jnp.bfloat16)
  attb1 = att1_p.astype(jnp.bfloat16)

  acc0 = spmm(h0, sdp, attb0, z_nd)
  out0, h1 = _dense(False, x_p, h0, acc0, combo, degd, dinv,
                    wpack(mlp_w0, mlp_b0), msg_w0.T, msg_b0[None, :],
                    skip_w0.T, skip_b0[None, :], jnp.zeros((d, d), jnp.float32),
                    jnp.zeros((1, d), jnp.float32), rb=rbn)

  acc1 = spmm(h1, sdp, attb1, z_nd)
  pred, _ = _dense(True, out0, h1, acc1, combo, degd, dinv,
                   wpack(mlp_w1, mlp_b1), msg_w1.T, msg_b1[None, :],
                   skip_w1.T, skip_b1[None, :], weight1.T, weight2, rb=rbn)

  lgrp = NW * 16
  lkt = -(-l // lgrp)
  l_pad = lgrp * lkt
  lp0 = jnp.concatenate(
      [label_edge_index[0], jnp.zeros((l_pad - l,), jnp.int32)])
  lp1 = jnp.concatenate(
      [label_edge_index[1], jnp.zeros((l_pad - l,), jnp.int32)])
  res = _make_decode(np_, l_pad)(pred.reshape(np_), lp0, lp1)
  return res[:l]


# speedup vs baseline: 1.5650x; 1.0097x over previous
import functools
import jax
import jax.numpy as jnp
from jax import lax
from jax.experimental import pallas as pl
from jax.experimental.pallas import tpu as pltpu
from jax.experimental.pallas import tpu_sc as plsc

NC = 2
NS = 16
NW = NC * NS
CH = 128

def _mesh():
  return plsc.VectorSubcoreMesh(core_axis_name="c", subcore_axis_name="s",
                                num_cores=NC, num_subcores=NS)


def _make_stats(n, kt):
  @functools.partial(
      pl.kernel,
      mesh=_mesh(),
      compiler_params=pltpu.CompilerParams(use_tc_tiling_on_sc=False),
      out_type=(
          jax.ShapeDtypeStruct((NC, n), jnp.float32),
          jax.ShapeDtypeStruct((NC, n), jnp.float32),
          jax.ShapeDtypeStruct((NC, n, 4), jnp.float32),
      ),
      scratch_types=[
          pltpu.VMEM_SHARED((n,), jnp.float32),
          pltpu.VMEM_SHARED((n,), jnp.float32),
          pltpu.VMEM_SHARED((n, 4), jnp.float32),
          pltpu.VMEM((kt, CH), jnp.int32),
          pltpu.VMEM((kt, CH), jnp.int32),
          pltpu.VMEM((kt, CH, 4), jnp.float32),
          pltpu.VMEM((kt, CH), jnp.float32),
      ],
  )
  def stats(src_h, dst_h, e_h, ones_h, z1_h, z4_h,
            degs_o, degd_o, esum_o,
            degs_sh, degd_sh, esum_sh, src_v, dst_v, e_v, ones_v):
    c = lax.axis_index("c")
    s = lax.axis_index("s")
    wid = c * NS + s
    rp = n // NS
    r0 = s * rp
    pltpu.sync_copy(z1_h.at[pl.ds(r0, rp)], degs_sh.at[pl.ds(r0, rp)])
    pltpu.sync_copy(z1_h.at[pl.ds(r0, rp)], degd_sh.at[pl.ds(r0, rp)])
    pltpu.sync_copy(z4_h.at[pl.ds(r0, rp)], esum_sh.at[pl.ds(r0, rp)])
    pltpu.sync_copy(src_h.at[pl.ds(wid * kt, kt)], src_v)
    pltpu.sync_copy(dst_h.at[pl.ds(wid * kt, kt)], dst_v)
    pltpu.sync_copy(e_h.at[pl.ds(wid * kt, kt)], e_v)
    pltpu.sync_copy(ones_h.at[pl.ds(wid * kt, kt)], ones_v)
    plsc.subcore_barrier()

    def chunk(j, carry):
      pltpu.sync_copy(ones_v.at[j], degs_sh.at[src_v.at[j]], add=True)
      pltpu.sync_copy(ones_v.at[j], degd_sh.at[dst_v.at[j]], add=True)
      pltpu.sync_copy(e_v.at[j], esum_sh.at[dst_v.at[j]], add=True)
      return carry

    lax.fori_loop(0, kt, chunk, 0)
    plsc.subcore_barrier()
    pltpu.sync_copy(degs_sh.at[pl.ds(r0, rp)], degs_o.at[c, pl.ds(r0, rp)])
    pltpu.sync_copy(degd_sh.at[pl.ds(r0, rp)], degd_o.at[c, pl.ds(r0, rp)])
    pltpu.sync_copy(esum_sh.at[pl.ds(r0, rp)], esum_o.at[c, pl.ds(r0, rp)])

  return stats


def _make_spmm(n, d, kt):
  d2 = d // 2

  @functools.partial(
      pl.kernel,
      mesh=_mesh(),
      compiler_params=pltpu.CompilerParams(use_tc_tiling_on_sc=False,
                                           needs_layout_passes=False),
      out_type=jax.ShapeDtypeStruct((n, d), jnp.float32),
      scratch_types=[
          pltpu.VMEM_SHARED((n, d2), jnp.float32),
          pltpu.VMEM_SHARED((n, d2), jnp.float32),
          pltpu.VMEM((kt, CH), jnp.int32),
          pltpu.VMEM((kt, CH), jnp.bfloat16),
          pltpu.VMEM((2, CH), jnp.int32),
          pltpu.VMEM((2, CH), jnp.int32),
          pltpu.VMEM((CH, d2), jnp.float32),
          pltpu.VMEM((CH, d2), jnp.float32),
          pltpu.SemaphoreType.DMA,
          pltpu.SemaphoreType.DMA,
          pltpu.SemaphoreType.DMA,
          pltpu.SemaphoreType.DMA,
      ],
  )
  def spmm(h_h, sdp_h, attb_h, z_h,
           acc_o,
           h_sh, acc_sh, sdp_v, attb_v, u0, u1, rows0, rows1,
           gsem0, gsem1, ssem0, ssem1):
    c = lax.axis_index("c")
    s = lax.axis_index("s")
    rp = n // NS
    r0 = s * rp
    c64 = c * d2
    pltpu.sync_copy(h_h.at[pl.ds(r0, rp), pl.ds(c64, d2)], h_sh.at[pl.ds(r0, rp)])
    pltpu.sync_copy(z_h.at[pl.ds(r0, rp), pl.ds(0, d2)], acc_sh.at[pl.ds(r0, rp)])
    cb = s * kt
    pltpu.sync_copy(sdp_h.at[pl.ds(cb, kt)], sdp_v)
    pltpu.sync_copy(attb_h.at[pl.ds(cb, kt)], attb_v)
    plsc.subcore_barrier()

    rows = (rows0, rows1)
    gsem = (gsem0, gsem1)
    ssem = (ssem0, ssem1)
    ubuf = (u0, u1)

    def unpack(j, ub):
      for g in range(CH // 16):
        v = sdp_v[j, pl.ds(g * 16, 16)]
        ub[0, pl.ds(g * 16, 16)] = v & 0xFFFF
        ub[1, pl.ds(g * 16, 16)] = lax.shift_right_logical(v, 16)

    unpack(0, u0)
    pltpu.async_copy(h_sh.at[u0.at[0]], rows0, gsem0)

    def pair(p, carry):
      for b in range(2):
        j = 2 * p + b
        rb, gs, ss, ub = rows[b], gsem[b], ssem[b], ubuf[b]
        nb, ngs, nss, nub = rows[1 - b], gsem[1 - b], ssem[1 - b], ubuf[1 - b]

        @pl.when(j + 1 < kt)
        def _():
          @pl.when(j >= 1)
          def _():
            pltpu.make_async_copy(nb, acc_sh.at[nub.at[1]], nss).wait()

          unpack(j + 1, nub)
          pltpu.async_copy(h_sh.at[nub.at[0]], nb, ngs)

        pltpu.make_async_copy(h_sh.at[ub.at[0]], rb, gs).wait()

        def grp32(g, cc):
          bv = plsc.bitcast(attb_v[j, pl.ds(g * 32, 32)], jnp.int32)
          for m in range(16):
            wm = bv[m]
            a0 = lax.bitcast_convert_type(lax.shift_left(wm, 16), jnp.float32)
            a1 = lax.bitcast_convert_type(wm & (-65536), jnp.float32)
            for t, a in ((0, a0), (1, a1)):
              i = g * 32 + 2 * m + t
              for q in range(d2 // 16):
                rb[i, pl.ds(q * 16, 16)] = rb[i, pl.ds(q * 16, 16)] * a
          return cc

        lax.fori_loop(0, CH // 32, grp32, 0)
        pltpu.async_copy(rb, acc_sh.at[ub.at[1]], ss, add=True)
      return carry

    lax.fori_loop(0, kt // 2, pair, 0)
    pltpu.make_async_copy(rows0, acc_sh.at[u0.at[1]], ssem0).wait()
    pltpu.make_async_copy(rows1, acc_sh.at[u1.at[1]], ssem1).wait()
    plsc.subcore_barrier()
    pltpu.sync_copy(acc_sh.at[pl.ds(r0, rp)],
                    acc_o.at[pl.ds(r0, rp), pl.ds(c64, d2)])

  return spmm


def _make_decode(n, lp):
  per = lp // NW

  @functools.partial(
      pl.kernel,
      mesh=_mesh(),
      compiler_params=pltpu.CompilerParams(use_tc_tiling_on_sc=False,
                                           needs_layout_passes=False),
      out_type=jax.ShapeDtypeStruct((lp,), jnp.float32),
      scratch_types=[
          pltpu.VMEM((n,), jnp.float32),
          pltpu.VMEM((per,), jnp.int32),
          pltpu.VMEM((per,), jnp.int32),
          pltpu.VMEM((per,), jnp.float32),
      ],
  )
  def decode(pred_h, li0_h, li1_h, out_o, pred_v, a_v, b_v, o_v):
    c = lax.axis_index("c")
    s = lax.axis_index("s")
    wid = c * NS + s
    pltpu.sync_copy(pred_h, pred_v)
    pltpu.sync_copy(li0_h.at[pl.ds(wid * per, per)], a_v)
    pltpu.sync_copy(li1_h.at[pl.ds(wid * per, per)], b_v)

    def grp(g, carry):
      ia = a_v[pl.ds(g * 16, 16)]
      ib = b_v[pl.ds(g * 16, 16)]
      va = plsc.load_gather(pred_v, [ia])
      vb = plsc.load_gather(pred_v, [ib])
      o_v[pl.ds(g * 16, 16)] = va * vb
      return carry

    lax.fori_loop(0, per // 16, grp, 0)
    pltpu.sync_copy(o_v, out_o.at[pl.ds(wid * per, per)])

  return decode


def _prelude_body(e_r, x, w, b, degs_p, degd_p, esum_p,
                  att_o, h0_o, dinv_o, degd_o, combo_o):
  deg = degs_p[0] + degs_p[1]
  dinv = jnp.where(deg > 0.0, lax.rsqrt(deg), 0.0)
  dinv_o[...] = dinv
  h0_o[...] = x[...] * dinv
  degd = degd_p[0] + degd_p[1]
  degd_o[...] = degd
  esum = esum_p[0] + esum_p[1]
  combo_o[...] = jnp.concatenate(
      [esum, degd, jnp.zeros_like(esum[:, :3])], axis=-1)
  logits = jnp.dot(e_r[...], w[...], preferred_element_type=jnp.float32)
  att_o[...] = jax.nn.sigmoid(logits + b[...])


def _prelude(e_r, x, w, b, degs_p, degd_p, esum_p, rb):
  n = x.shape[0]
  grid = n // rb
  return pl.pallas_call(
      _prelude_body,
      grid=(grid,),
      in_specs=[
          pl.BlockSpec((rb, e_r.shape[1]), lambda i: (i, 0)),
          pl.BlockSpec((rb, x.shape[1]), lambda i: (i, 0)),
          pl.BlockSpec(w.shape, lambda i: (0, 0)),
          pl.BlockSpec(b.shape, lambda i: (0, 0)),
          pl.BlockSpec((NC, rb, 1), lambda i: (0, i, 0)),
          pl.BlockSpec((NC, rb, 1), lambda i: (0, i, 0)),
          pl.BlockSpec((NC, rb, 4), lambda i: (0, i, 0)),
      ],
      out_specs=[
          pl.BlockSpec((rb, w.shape[1]), lambda i: (i, 0)),
          pl.BlockSpec((rb, x.shape[1]), lambda i: (i, 0)),
          pl.BlockSpec((rb, 1), lambda i: (i, 0)),
          pl.BlockSpec((rb, 1), lambda i: (i, 0)),
          pl.BlockSpec((rb, 8), lambda i: (i, 0)),
      ],
      out_shape=[
          jax.ShapeDtypeStruct((n, w.shape[1]), jnp.float32),
          jax.ShapeDtypeStruct((n, x.shape[1]), jnp.float32),
          jax.ShapeDtypeStruct((n, 1), jnp.float32),
          jax.ShapeDtypeStruct((n, 1), jnp.float32),
          jax.ShapeDtypeStruct((n, 8), jnp.float32),
      ],
  )(e_r, x, w, b, degs_p, degd_p, esum_p)


def _dense_body(final, x, h, acc_p, combo, degd, dinv, wc, msg_wt, msg_b,
                skip_wt, skip_b, w1t, w2v, out_o, hn_o):
  acc = acc_p[...]
  rst = acc + jnp.dot(combo[...], wc[...], preferred_element_type=jnp.float32)
  rst = rst + degd[...] * h[...]
  rst = rst * dinv[...]
  out = (jnp.dot(rst, msg_wt[...], preferred_element_type=jnp.float32) + msg_b[...]
         + jnp.dot(x[...], skip_wt[...], preferred_element_type=jnp.float32)
         + skip_b[...])
  if not final:
    out_o[...] = out
    hn_o[...] = out * dinv[...]
  else:
    nrm = jnp.maximum(jnp.sqrt(jnp.sum(out * out, axis=-1, keepdims=True)), 1e-12)
    hn = out / nrm
    p1 = jnp.maximum(
        jnp.dot(hn, w1t[...], preferred_element_type=jnp.float32), 0.0)
    pred = jax.nn.sigmoid(jnp.sum(p1 * w2v[...], axis=-1, keepdims=True))
    out_o[...] = pred
    hn_o[...] = hn


def _dense(final, x, h, acc_p, combo, degd, dinv, wc, msg_wt, msg_b,
           skip_wt, skip_b, w1t, w2v, rb):
  n, d = x.shape
  grid = n // rb
  full = lambda a: pl.BlockSpec(a.shape, lambda i: tuple(0 for _ in a.shape))
  out_w = 1 if final else d
  return pl.pallas_call(
      functools.partial(_dense_body, final),
      grid=(grid,),
      in_specs=[
          pl.BlockSpec((rb, d), lambda i: (i, 0)),
          pl.BlockSpec((rb, d), lambda i: (i, 0)),
          pl.BlockSpec((rb, d), lambda i: (i, 0)),
          pl.BlockSpec((rb, 8), lambda i: (i, 0)),
          pl.BlockSpec((rb, 1), lambda i: (i, 0)),
          pl.BlockSpec((rb, 1), lambda i: (i, 0)),
          full(wc), full(msg_wt), full(msg_b), full(skip_wt), full(skip_b),
          full(w1t), full(w2v),
      ],
      out_specs=[
          pl.BlockSpec((rb, out_w), lambda i: (i, 0)),
          pl.BlockSpec((rb, d), lambda i: (i, 0)),
      ],
      out_shape=[
          jax.ShapeDtypeStruct((n, out_w), jnp.float32),
          jax.ShapeDtypeStruct((n, d), jnp.float32),
      ],
  )(x, h, acc_p, combo, degd, dinv, wc, msg_wt, msg_b, skip_wt, skip_b,
    w1t, w2v)


def kernel(x, e, edge_index, label_edge_index,
           skip_w0, skip_b0, msg_w0, msg_b0, mlp_w0, mlp_b0, att_w0, att_b0,
           skip_w1, skip_b1, msg_w1, msg_b1, mlp_w1, mlp_b1, att_w1, att_b1,
           weight1, weight2):
  n, d = x.shape
  eN, de = e.shape
  l = label_edge_index.shape[1]

  src = edge_index[0]
  dst = edge_index[1]

  rbn = 1024
  np_ = -(-n // rbn) * rbn
  x_p = jnp.concatenate([x, jnp.zeros((np_ - n, d), jnp.float32)])

  grp = NW * CH
  kt = -(-(-(-eN // grp)) // 8) * 8
  kt2 = 2 * kt
  e_pad = grp * kt
  pad = e_pad - eN
  src_p = jnp.concatenate([src, jnp.zeros((pad,), jnp.int32)]).reshape(-1, CH)
  dst_p = jnp.concatenate([dst, jnp.zeros((pad,), jnp.int32)]).reshape(-1, CH)
  e_p = jnp.concatenate([e, jnp.zeros((pad, de), jnp.float32)]).reshape(-1, CH, de)
  ones_p = jnp.concatenate(
      [jnp.ones((eN,), jnp.float32), jnp.zeros((pad,), jnp.float32)]
  ).reshape(-1, CH)
  z_n1 = jnp.zeros((np_,), jnp.float32)
  z_n4 = jnp.zeros((np_, 4), jnp.float32)
  z_nd = jnp.zeros((np_, d), jnp.float32)

  degs_p, degd_p, esum_p = _make_stats(np_, kt)(src_p, dst_p, e_p, ones_p,
                                                z_n1, z_n4)
  degs_p = degs_p.reshape(NC, np_, 1)
  degd_p = degd_p.reshape(NC, np_, 1)

  epr = 128 // de
  ner = eN // epr
  e_r = e.reshape(ner, epr * de)
  e_r = jnp.concatenate(
      [e_r, jnp.zeros((np_ - ner, epr * de), jnp.float32)])
  wa = jnp.concatenate(
      [jnp.kron(jnp.eye(epr, dtype=jnp.float32), att_w0.reshape(de, 1)),
       jnp.kron(jnp.eye(epr, dtype=jnp.float32), att_w1.reshape(de, 1))],
      axis=1)
  ba = jnp.concatenate([jnp.tile(att_b0, epr), jnp.tile(att_b1, epr)])[None, :]
  att2, h0, dinv, degd, combo = _prelude(e_r, x_p, wa, ba, degs_p, degd_p,
                                         esum_p, rb=rbn)
  att0 = att2[:ner, :epr].reshape(eN)
  att1 = att2[:ner, epr:].reshape(eN)
  zpad = jnp.zeros((pad,), jnp.float32)
  att0_p = jnp.concatenate([att0, zpad]).reshape(-1, CH)
  att1_p = jnp.concatenate([att1, zpad]).reshape(-1, CH)

  def wpack(mlp_w, mlp_b):
    return jnp.concatenate(
        [mlp_w.T, mlp_b[None, :], jnp.zeros((3, d), jnp.float32)], axis=0)

  spmm = _make_spmm(np_, d, kt2)
  sdp = (src_p + dst_p * 65536).astype(jnp.int32)
  attb0 = att0_p.astype(jnp.bfloat16)
  attb1 = att1_p.astype(jnp.bfloat16)

  acc0 = spmm(h0, sdp, attb0, z_nd)
  out0, h1 = _dense(False, x_p, h0, acc0, combo, degd, dinv,
                    wpack(mlp_w0, mlp_b0), msg_w0.T, msg_b0[None, :],
                    skip_w0.T, skip_b0[None, :], jnp.zeros((d, d), jnp.float32),
                    jnp.zeros((1, d), jnp.float32), rb=rbn)

  acc1 = spmm(h1, sdp, attb1, z_nd)
  pred, _ = _dense(True, out0, h1, acc1, combo, degd, dinv,
                   wpack(mlp_w1, mlp_b1), msg_w1.T, msg_b1[None, :],
                   skip_w1.T, skip_b1[None, :], weight1.T, weight2, rb=rbn)

  lgrp = NW * 16
  lkt = -(-l // lgrp)
  l_pad = lgrp * lkt
  lp0 = jnp.concatenate(
      [label_edge_index[0], jnp.zeros((l_pad - l,), jnp.int32)])
  lp1 = jnp.concatenate(
      [label_edge_index[1], jnp.zeros((l_pad - l,), jnp.int32)])
  res = _make_decode(np_, l_pad)(pred.reshape(np_), lp0, lp1)
  return res[:l]
